# baseline (speedup 1.0000x reference)
import jax
import jax.numpy as jnp
from jax.experimental import pallas as pl
from jax.experimental.pallas import tpu as pltpu

T = 2048
D = 1024
H = 16
KV = 8
HD = 64
E = 8
K = 2
F = 512
THETA = 1000000.0
EPS = 1e-6
BT = 256


def _rmsnorm(x, w):
    var = jnp.mean(x * x, axis=-1, keepdims=True)
    return x * jax.lax.rsqrt(var + EPS) * w


def _rope(x, cos, sin, nh):
    parts = []
    for h in range(nh):
        x1 = x[:, h * HD : h * HD + HD // 2]
        x2 = x[:, h * HD + HD // 2 : (h + 1) * HD]
        parts.append(x2)
        parts.append(x1)
    xs = jnp.concatenate(parts, axis=1)
    cosf = jnp.concatenate([cos] * nh, axis=1)
    sinf = jnp.concatenate([sin] * nh, axis=1)
    return x * cosf + xs * sinf


def _qkv_body(hid_ref, ln1_ref, cos_ref, sin_ref, wq_ref, wk_ref, wv_ref,
              bq_ref, bk_ref, bv_ref, qo_ref, ko_ref, vo_ref):
    x = hid_ref[...]
    xn = _rmsnorm(x, ln1_ref[...]).astype(jnp.bfloat16)
    cos = cos_ref[...]
    sin = sin_ref[...]
    q = jnp.dot(xn, wq_ref[...], preferred_element_type=jnp.float32) + bq_ref[...]
    k = jnp.dot(xn, wk_ref[...], preferred_element_type=jnp.float32) + bk_ref[...]
    v = jnp.dot(xn, wv_ref[...], preferred_element_type=jnp.float32) + bv_ref[...]
    qo_ref[...] = _rope(q, cos, sin, H).astype(jnp.bfloat16)
    ko_ref[...] = _rope(k, cos, sin, KV).astype(jnp.bfloat16)
    vo_ref[...] = v.astype(jnp.bfloat16)


def _attn_body(q_ref, k_ref, v_ref, o_ref, acc_ref, m_ref, l_ref):
    qi = pl.program_id(0)
    qb = q_ref[...]
    acc_ref[...] = jnp.zeros((BT, H * HD), jnp.float32)
    m_ref[...] = jnp.full((BT, 128), -1e30, jnp.float32)
    l_ref[...] = jnp.zeros((BT, 128), jnp.float32)
    rows = qi * BT + jax.lax.broadcasted_iota(jnp.int32, (BT, BT), 0)
    for j in range(T // BT):
        @pl.when(j <= qi)
        def _():
            kj = k_ref[j * BT : (j + 1) * BT, :]
            vj = v_ref[j * BT : (j + 1) * BT, :]
            cols = j * BT + jax.lax.broadcasted_iota(jnp.int32, (BT, BT), 1)
            causal = rows >= cols
            for h in range(H):
                qh = qb[:, h * HD : (h + 1) * HD]
                kvh = h // (H // KV)
                kh = kj[:, kvh * HD : (kvh + 1) * HD]
                vh = vj[:, kvh * HD : (kvh + 1) * HD]
                s = jax.lax.dot_general(qh, kh, (((1,), (1,)), ((), ())),
                                        preferred_element_type=jnp.float32)
                s = s * (HD ** -0.5)
                s = jnp.where(causal, s, -1e30)
                m_old = m_ref[:, h : h + 1]
                m_new = jnp.maximum(m_old, jnp.max(s, axis=-1, keepdims=True))
                p = jnp.exp(s - m_new)
                corr = jnp.exp(m_old - m_new)
                l_ref[:, h : h + 1] = (l_ref[:, h : h + 1] * corr
                                       + jnp.sum(p, axis=-1, keepdims=True))
                pv = jnp.dot(p.astype(jnp.bfloat16), vh,
                             preferred_element_type=jnp.float32)
                acc_ref[:, h * HD : (h + 1) * HD] = (
                    acc_ref[:, h * HD : (h + 1) * HD] * corr + pv)
                m_ref[:, h : h + 1] = m_new
    linv = 1.0 / l_ref[:, :H]
    lfull = jnp.concatenate(
        [jax.lax.broadcast_in_dim(linv[:, h : h + 1], (BT, HD), (0, 1))
         for h in range(H)], axis=1)
    o_ref[...] = (acc_ref[...] * lfull).astype(jnp.bfloat16)


def _post_body(attn_ref, wo_ref, hid_ref, ln2_ref, wg_ref,
               h_ref, x2_ref, comb_ref):
    a = attn_ref[...]
    ho = jnp.dot(a, wo_ref[...], preferred_element_type=jnp.float32)
    h = hid_ref[...] + ho
    h_ref[...] = h
    x2 = _rmsnorm(h, ln2_ref[...])
    x2_ref[...] = x2.astype(jnp.bfloat16)
    logits = jnp.dot(x2, wg_ref[...], preferred_element_type=jnp.float32)
    mx = jnp.max(logits, axis=-1, keepdims=True)
    pr = jnp.exp(logits - mx)
    pr = pr / jnp.sum(pr, axis=-1, keepdims=True)
    lanes = jax.lax.broadcasted_iota(jnp.int32, (BT, E), 1)
    m1 = jnp.max(pr, axis=-1, keepdims=True)
    idx1 = jnp.min(jnp.where(pr == m1, lanes, 127), axis=-1, keepdims=True)
    sel1 = lanes == idx1
    pr2 = jnp.where(sel1, -1.0, pr)
    m2 = jnp.max(pr2, axis=-1, keepdims=True)
    idx2 = jnp.min(jnp.where(pr2 == m2, lanes, 127), axis=-1, keepdims=True)
    sel2 = lanes == idx2
    comb = (jnp.where(sel1, m1, 0.0) + jnp.where(sel2, m2, 0.0)) / (m1 + m2)
    comb_ref[...] = comb


def _moe_body(x2_ref, h_ref, comb_ref, wg_ref, wu_ref, wd_ref, o_ref):
    xb = x2_ref[...]
    acc = h_ref[...]
    comb = comb_ref[...]
    for e in range(E):
        g = jnp.dot(xb, wg_ref[e], preferred_element_type=jnp.float32)
        u = jnp.dot(xb, wu_ref[e], preferred_element_type=jnp.float32)
        inter = (g * jax.lax.logistic(g) * u).astype(jnp.bfloat16)
        y = jnp.dot(inter, wd_ref[e], preferred_element_type=jnp.float32)
        acc = acc + y * comb[:, e : e + 1]
    o_ref[...] = acc


def kernel(positions, hidden_states, ln1_w, ln2_w, Wq, bq, Wk, bk, Wv, bv,
           Wo, Wg, w_gate, w_up, w_down):
    f32 = jnp.float32
    bf16 = jnp.bfloat16
    half = HD // 2
    inv_freq = 1.0 / (THETA ** (jnp.arange(0, half, dtype=f32) / half))
    freqs = positions.astype(f32)[:, None] * inv_freq[None, :]
    c = jnp.cos(freqs)
    s = jnp.sin(freqs)
    cosA = jnp.concatenate([c, c], axis=1)
    sinA = jnp.concatenate([-s, s], axis=1)

    ln1 = ln1_w.reshape(1, D)
    ln2 = ln2_w.reshape(1, D)
    bq2 = bq.reshape(1, H * HD)
    bk2 = bk.reshape(1, KV * HD)
    bv2 = bv.reshape(1, KV * HD)
    wq_b = Wq.astype(bf16)
    wk_b = Wk.astype(bf16)
    wv_b = Wv.astype(bf16)
    wo_b = Wo.astype(bf16)
    wgate_b = w_gate.astype(bf16)
    wup_b = w_up.astype(bf16)
    wdown_b = w_down.astype(bf16)

    nt = T // BT
    q, k, v = pl.pallas_call(
        _qkv_body,
        grid=(nt,),
        in_specs=[
            pl.BlockSpec((BT, D), lambda i: (i, 0)),
            pl.BlockSpec((1, D), lambda i: (0, 0)),
            pl.BlockSpec((BT, HD), lambda i: (i, 0)),
            pl.BlockSpec((BT, HD), lambda i: (i, 0)),
            pl.BlockSpec((D, H * HD), lambda i: (0, 0)),
            pl.BlockSpec((D, KV * HD), lambda i: (0, 0)),
            pl.BlockSpec((D, KV * HD), lambda i: (0, 0)),
            pl.BlockSpec((1, H * HD), lambda i: (0, 0)),
            pl.BlockSpec((1, KV * HD), lambda i: (0, 0)),
            pl.BlockSpec((1, KV * HD), lambda i: (0, 0)),
        ],
        out_specs=[
            pl.BlockSpec((BT, H * HD), lambda i: (i, 0)),
            pl.BlockSpec((BT, KV * HD), lambda i: (i, 0)),
            pl.BlockSpec((BT, KV * HD), lambda i: (i, 0)),
        ],
        out_shape=[
            jax.ShapeDtypeStruct((T, H * HD), bf16),
            jax.ShapeDtypeStruct((T, KV * HD), bf16),
            jax.ShapeDtypeStruct((T, KV * HD), bf16),
        ],
    )(hidden_states, ln1, cosA, sinA, wq_b, wk_b, wv_b, bq2, bk2, bv2)

    attn = pl.pallas_call(
        _attn_body,
        grid=(nt,),
        in_specs=[
            pl.BlockSpec((BT, H * HD), lambda i: (i, 0)),
            pl.BlockSpec((T, KV * HD), lambda i: (0, 0)),
            pl.BlockSpec((T, KV * HD), lambda i: (0, 0)),
        ],
        out_specs=pl.BlockSpec((BT, H * HD), lambda i: (i, 0)),
        out_shape=jax.ShapeDtypeStruct((T, H * HD), bf16),
        scratch_shapes=[
            pltpu.VMEM((BT, H * HD), f32),
            pltpu.VMEM((BT, 128), f32),
            pltpu.VMEM((BT, 128), f32),
        ],
    )(q, k, v)

    h, x2b, comb = pl.pallas_call(
        _post_body,
        grid=(nt,),
        in_specs=[
            pl.BlockSpec((BT, H * HD), lambda i: (i, 0)),
            pl.BlockSpec((H * HD, D), lambda i: (0, 0)),
            pl.BlockSpec((BT, D), lambda i: (i, 0)),
            pl.BlockSpec((1, D), lambda i: (0, 0)),
            pl.BlockSpec((D, E), lambda i: (0, 0)),
        ],
        out_specs=[
            pl.BlockSpec((BT, D), lambda i: (i, 0)),
            pl.BlockSpec((BT, D), lambda i: (i, 0)),
            pl.BlockSpec((BT, E), lambda i: (i, 0)),
        ],
        out_shape=[
            jax.ShapeDtypeStruct((T, D), f32),
            jax.ShapeDtypeStruct((T, D), bf16),
            jax.ShapeDtypeStruct((T, E), f32),
        ],
    )(attn, wo_b, hidden_states, ln2, Wg)

    out = pl.pallas_call(
        _moe_body,
        grid=(nt,),
        in_specs=[
            pl.BlockSpec((BT, D), lambda i: (i, 0)),
            pl.BlockSpec((BT, D), lambda i: (i, 0)),
            pl.BlockSpec((BT, E), lambda i: (i, 0)),
            pl.BlockSpec((E, D, F), lambda i: (0, 0, 0)),
            pl.BlockSpec((E, D, F), lambda i: (0, 0, 0)),
            pl.BlockSpec((E, F, D), lambda i: (0, 0, 0)),
        ],
        out_specs=pl.BlockSpec((BT, D), lambda i: (i, 0)),
        out_shape=jax.ShapeDtypeStruct((T, D), f32),
    )(x2b, h, comb, wgate_b, wup_b, wdown_b)

    return out


# speedup vs baseline: 2.5226x; 2.5226x over previous
import functools

import jax
import jax.numpy as jnp
from jax.experimental import pallas as pl
from jax.experimental.pallas import tpu as pltpu

T = 2048
D = 1024
H = 16
KV = 8
HD = 64
E = 8
K = 2
F = 512
THETA = 1000000.0
EPS = 1e-6
BT = 256


def _rmsnorm(x, w):
    var = jnp.mean(x * x, axis=-1, keepdims=True)
    return x * jax.lax.rsqrt(var + EPS) * w


def _rope(x, cos, sin, nh):
    parts = []
    for h in range(nh):
        x1 = x[:, h * HD : h * HD + HD // 2]
        x2 = x[:, h * HD + HD // 2 : (h + 1) * HD]
        parts.append(x2)
        parts.append(x1)
    xs = jnp.concatenate(parts, axis=1)
    cosf = jnp.concatenate([cos] * nh, axis=1)
    sinf = jnp.concatenate([sin] * nh, axis=1)
    return x * cosf + xs * sinf


def _qkv_body(hid_ref, ln1_ref, cos_ref, sin_ref, wq_ref, wk_ref, wv_ref,
              bq_ref, bk_ref, bv_ref, qo_ref, ko_ref, vo_ref):
    x = hid_ref[...]
    xn = _rmsnorm(x, ln1_ref[...]).astype(jnp.bfloat16)
    cos = cos_ref[...]
    sin = sin_ref[...]
    q = jnp.dot(xn, wq_ref[...], preferred_element_type=jnp.float32) + bq_ref[...]
    k = jnp.dot(xn, wk_ref[...], preferred_element_type=jnp.float32) + bk_ref[...]
    v = jnp.dot(xn, wv_ref[...], preferred_element_type=jnp.float32) + bv_ref[...]
    qo_ref[...] = _rope(q, cos, sin, H).astype(jnp.bfloat16)
    ko_ref[...] = _rope(k, cos, sin, KV).astype(jnp.bfloat16)
    vo_ref[...] = v.astype(jnp.bfloat16)


def _attn_body(q_ref, k_ref, v_ref, o_ref, *, kext, qbase):
    qi = pl.program_id(0)
    rows = (qbase + qi) * BT + jax.lax.broadcasted_iota(jnp.int32, (BT, kext), 0)
    cols = jax.lax.broadcasted_iota(jnp.int32, (BT, kext), 1)
    causal = rows >= cols
    qb = q_ref[...]
    kb = k_ref[...]
    vb = v_ref[...]
    outs = []
    for h in range(H):
        qh = qb[:, h * HD : (h + 1) * HD]
        kvh = h // (H // KV)
        kh = kb[:, kvh * HD : (kvh + 1) * HD]
        vh = vb[:, kvh * HD : (kvh + 1) * HD]
        s = jax.lax.dot_general(qh, kh, (((1,), (1,)), ((), ())),
                                preferred_element_type=jnp.float32)
        s = s * (HD ** -0.5)
        s = jnp.where(causal, s, -1e30)
        m = jnp.max(s, axis=-1, keepdims=True)
        p = jnp.exp(s - m)
        denom = jnp.sum(p, axis=-1, keepdims=True)
        o = jnp.dot(p.astype(jnp.bfloat16), vh,
                    preferred_element_type=jnp.float32)
        outs.append((o / denom).astype(jnp.bfloat16))
    o_ref[...] = jnp.concatenate(outs, axis=1)


def _post_body(attn_ref, wo_ref, hid_ref, ln2_ref, wg_ref,
               h_ref, x2_ref, comb_ref):
    a = attn_ref[...]
    ho = jnp.dot(a, wo_ref[...], preferred_element_type=jnp.float32)
    h = hid_ref[...] + ho
    h_ref[...] = h
    x2 = _rmsnorm(h, ln2_ref[...])
    x2_ref[...] = x2.astype(jnp.bfloat16)
    logits = jnp.dot(x2, wg_ref[...], preferred_element_type=jnp.float32)
    mx = jnp.max(logits, axis=-1, keepdims=True)
    pr = jnp.exp(logits - mx)
    pr = pr / jnp.sum(pr, axis=-1, keepdims=True)
    lanes = jax.lax.broadcasted_iota(jnp.int32, (BT, E), 1)
    m1 = jnp.max(pr, axis=-1, keepdims=True)
    idx1 = jnp.min(jnp.where(pr == m1, lanes, 127), axis=-1, keepdims=True)
    sel1 = lanes == idx1
    pr2 = jnp.where(sel1, -1.0, pr)
    m2 = jnp.max(pr2, axis=-1, keepdims=True)
    idx2 = jnp.min(jnp.where(pr2 == m2, lanes, 127), axis=-1, keepdims=True)
    sel2 = lanes == idx2
    comb = (jnp.where(sel1, m1, 0.0) + jnp.where(sel2, m2, 0.0)) / (m1 + m2)
    comb_ref[...] = comb


def _moe_body(x2_ref, h_ref, comb_ref, wg_ref, wu_ref, wd_ref, o_ref):
    xb = x2_ref[...]
    acc = h_ref[...]
    comb = comb_ref[...]
    for e in range(E):
        g = jnp.dot(xb, wg_ref[e], preferred_element_type=jnp.float32)
        u = jnp.dot(xb, wu_ref[e], preferred_element_type=jnp.float32)
        inter = (g * jax.lax.logistic(g) * u).astype(jnp.bfloat16)
        y = jnp.dot(inter, wd_ref[e], preferred_element_type=jnp.float32)
        acc = acc + y * comb[:, e : e + 1]
    o_ref[...] = acc


def kernel(positions, hidden_states, ln1_w, ln2_w, Wq, bq, Wk, bk, Wv, bv,
           Wo, Wg, w_gate, w_up, w_down):
    f32 = jnp.float32
    bf16 = jnp.bfloat16
    half = HD // 2
    inv_freq = 1.0 / (THETA ** (jnp.arange(0, half, dtype=f32) / half))
    freqs = positions.astype(f32)[:, None] * inv_freq[None, :]
    c = jnp.cos(freqs)
    s = jnp.sin(freqs)
    cosA = jnp.concatenate([c, c], axis=1)
    sinA = jnp.concatenate([-s, s], axis=1)

    ln1 = ln1_w.reshape(1, D)
    ln2 = ln2_w.reshape(1, D)
    bq2 = bq.reshape(1, H * HD)
    bk2 = bk.reshape(1, KV * HD)
    bv2 = bv.reshape(1, KV * HD)
    wq_b = Wq.astype(bf16)
    wk_b = Wk.astype(bf16)
    wv_b = Wv.astype(bf16)
    wo_b = Wo.astype(bf16)
    wgate_b = w_gate.astype(bf16)
    wup_b = w_up.astype(bf16)
    wdown_b = w_down.astype(bf16)

    nt = T // BT
    q, k, v = pl.pallas_call(
        _qkv_body,
        grid=(nt,),
        in_specs=[
            pl.BlockSpec((BT, D), lambda i: (i, 0)),
            pl.BlockSpec((1, D), lambda i: (0, 0)),
            pl.BlockSpec((BT, HD), lambda i: (i, 0)),
            pl.BlockSpec((BT, HD), lambda i: (i, 0)),
            pl.BlockSpec((D, H * HD), lambda i: (0, 0)),
            pl.BlockSpec((D, KV * HD), lambda i: (0, 0)),
            pl.BlockSpec((D, KV * HD), lambda i: (0, 0)),
            pl.BlockSpec((1, H * HD), lambda i: (0, 0)),
            pl.BlockSpec((1, KV * HD), lambda i: (0, 0)),
            pl.BlockSpec((1, KV * HD), lambda i: (0, 0)),
        ],
        out_specs=[
            pl.BlockSpec((BT, H * HD), lambda i: (i, 0)),
            pl.BlockSpec((BT, KV * HD), lambda i: (i, 0)),
            pl.BlockSpec((BT, KV * HD), lambda i: (i, 0)),
        ],
        out_shape=[
            jax.ShapeDtypeStruct((T, H * HD), bf16),
            jax.ShapeDtypeStruct((T, KV * HD), bf16),
            jax.ShapeDtypeStruct((T, KV * HD), bf16),
        ],
    )(hidden_states, ln1, cosA, sinA, wq_b, wk_b, wv_b, bq2, bk2, bv2)

    attn_parts = []
    GRP = 2
    for g in range(nt // GRP):
        qbase = g * GRP
        kext = (qbase + GRP) * BT
        part = pl.pallas_call(
            functools.partial(_attn_body, kext=kext, qbase=qbase),
            grid=(GRP,),
            in_specs=[
                pl.BlockSpec((BT, H * HD), lambda i, qb=qbase: (qb + i, 0)),
                pl.BlockSpec((kext, KV * HD), lambda i: (0, 0)),
                pl.BlockSpec((kext, KV * HD), lambda i: (0, 0)),
            ],
            out_specs=pl.BlockSpec((BT, H * HD), lambda i: (i, 0)),
            out_shape=jax.ShapeDtypeStruct((GRP * BT, H * HD), bf16),
        )(q, k, v)
        attn_parts.append(part)
    attn = jnp.concatenate(attn_parts, axis=0)

    h, x2b, comb = pl.pallas_call(
        _post_body,
        grid=(nt,),
        in_specs=[
            pl.BlockSpec((BT, H * HD), lambda i: (i, 0)),
            pl.BlockSpec((H * HD, D), lambda i: (0, 0)),
            pl.BlockSpec((BT, D), lambda i: (i, 0)),
            pl.BlockSpec((1, D), lambda i: (0, 0)),
            pl.BlockSpec((D, E), lambda i: (0, 0)),
        ],
        out_specs=[
            pl.BlockSpec((BT, D), lambda i: (i, 0)),
            pl.BlockSpec((BT, D), lambda i: (i, 0)),
            pl.BlockSpec((BT, E), lambda i: (i, 0)),
        ],
        out_shape=[
            jax.ShapeDtypeStruct((T, D), f32),
            jax.ShapeDtypeStruct((T, D), bf16),
            jax.ShapeDtypeStruct((T, E), f32),
        ],
    )(attn, wo_b, hidden_states, ln2, Wg)

    out = pl.pallas_call(
        _moe_body,
        grid=(nt,),
        in_specs=[
            pl.BlockSpec((BT, D), lambda i: (i, 0)),
            pl.BlockSpec((BT, D), lambda i: (i, 0)),
            pl.BlockSpec((BT, E), lambda i: (i, 0)),
            pl.BlockSpec((E, D, F), lambda i: (0, 0, 0)),
            pl.BlockSpec((E, D, F), lambda i: (0, 0, 0)),
            pl.BlockSpec((E, F, D), lambda i: (0, 0, 0)),
        ],
        out_specs=pl.BlockSpec((BT, D), lambda i: (i, 0)),
        out_shape=jax.ShapeDtypeStruct((T, D), f32),
    )(x2b, h, comb, wgate_b, wup_b, wdown_b)

    return out
